# baseline (speedup 1.0000x reference)
import functools

import jax
import jax.numpy as jnp
from jax import lax
from jax.experimental import pallas as pl
from jax.experimental.pallas import tpu as pltpu

B = 1024
V = 100000
MARGIN = 0.3
SCALE = 32.0

LOG2E = 1.4426950408889634
LN2 = 0.6931471805599453
K2 = SCALE * LOG2E

RBLK = 32
GRID_R = 1
NI = B // RBLK // GRID_R


TCUT = V - 128


def _tc_kernel(cos_ref, lab_ref, out_ref, acc_ref):
    i = pl.program_id(1)

    @pl.when(i == 0)
    def _init():
        acc_ref[...] = jnp.zeros((1, 1), jnp.float32)

    x2 = cos_ref[...].astype(jnp.float32) * K2
    bm = jnp.max(x2, axis=1, keepdims=True)
    s = jnp.sum(jnp.exp2(x2 - bm), axis=1, keepdims=True)
    lse2 = bm + jnp.log2(s)

    segs = []
    lanes = []
    tlanes = []
    for j in range(RBLK):
        lab = lab_ref[j, 0]
        safe = jnp.minimum(lab, TCUT - 1)
        start = pl.multiple_of((safe >> 7) << 7, 128)
        segs.append(cos_ref[pl.ds(j, 1), pl.ds(start, 128)])
        lanes.append(jnp.where(lab < TCUT, lab & 127, -1).reshape(1, 1))
        tlanes.append(jnp.where(lab < TCUT, -1, lab - TCUT).reshape(1, 1))
    seg = jnp.concatenate(segs, axis=0).astype(jnp.float32)
    lane = jnp.concatenate(lanes, axis=0)
    tlane = jnp.concatenate(tlanes, axis=0)
    segfix = cos_ref[:, TCUT:V].astype(jnp.float32)
    lj = lax.broadcasted_iota(jnp.int32, (RBLK, 128), 1)
    cl = jnp.sum(
        jnp.where(lj == lane, seg, 0.0), axis=1, keepdims=True
    ) + jnp.sum(
        jnp.where(lj == tlane, segfix, 0.0), axis=1, keepdims=True
    )

    t = jnp.exp2(cl * K2 - lse2)
    d = 2.0 ** (-MARGIN * SCALE * LOG2E)
    term = jnp.maximum(1.0 - t * (1.0 - d), 1e-37)
    lse2m = lse2 + jnp.log2(term)
    loss = LN2 * lse2m - SCALE * (cl - MARGIN)
    acc_ref[...] += jnp.sum(loss, axis=0, keepdims=True)

    @pl.when(i == NI - 1)
    def _finish():
        out_ref[0, :, :] = acc_ref[...] * (1.0 / B)


@functools.partial(jax.jit, static_argnames=("interpret",))
def _amloss(cosine, label, interpret=False):
    lab2d = label.reshape(B, 1).astype(jnp.int32)
    cos16 = cosine.astype(jnp.bfloat16)
    out = pl.pallas_call(
        _tc_kernel,
        grid=(GRID_R, NI),
        in_specs=[
            pl.BlockSpec((RBLK, V), lambda r, i: (r * NI + i, 0)),
            pl.BlockSpec(
                (RBLK, 1),
                lambda r, i: (r * NI + i, 0),
                memory_space=pltpu.SMEM,
            ),
        ],
        out_specs=pl.BlockSpec((1, 1, 1), lambda r, i: (r, 0, 0)),
        out_shape=jax.ShapeDtypeStruct((GRID_R, 1, 1), jnp.float32),
        scratch_shapes=[
            pltpu.VMEM((1, 1), jnp.float32),
        ],
        compiler_params=pltpu.CompilerParams(
            dimension_semantics=("parallel", "arbitrary")
        ),
        interpret=interpret,
    )(cos16, lab2d)
    return jnp.sum(out)


def kernel(cosine, label):
    return _amloss(cosine, label)


# speedup vs baseline: 1.1700x; 1.1700x over previous
import functools

import jax
import jax.numpy as jnp
from jax import lax
from jax.experimental import pallas as pl
from jax.experimental.pallas import tpu as pltpu

B = 1024
V = 100000
MARGIN = 0.3
SCALE = 32.0

LOG2E = 1.4426950408889634
LN2 = 0.6931471805599453
K2 = SCALE * LOG2E

RBLK = 32
GRID_R = 1
NI = B // RBLK // GRID_R


TCUT = V - 128


def _tc_kernel(cos_ref, lab_ref, out_ref, acc_ref):
    i = pl.program_id(1)

    @pl.when(i == 0)
    def _init():
        acc_ref[...] = jnp.zeros((1, 1), jnp.float32)

    x2 = cos_ref[...] * K2
    bm = jnp.max(x2, axis=1, keepdims=True)
    s = jnp.sum(jnp.exp2(x2 - bm), axis=1, keepdims=True)
    lse2 = bm + jnp.log2(s)

    segs = []
    lanes = []
    tlanes = []
    for j in range(RBLK):
        lab = lab_ref[j, 0]
        safe = jnp.minimum(lab, TCUT - 1)
        start = pl.multiple_of((safe >> 7) << 7, 128)
        segs.append(cos_ref[pl.ds(j, 1), pl.ds(start, 128)])
        lanes.append(jnp.where(lab < TCUT, lab & 127, -1).reshape(1, 1))
        tlanes.append(jnp.where(lab < TCUT, -1, lab - TCUT).reshape(1, 1))
    seg = jnp.concatenate(segs, axis=0)
    lane = jnp.concatenate(lanes, axis=0)
    tlane = jnp.concatenate(tlanes, axis=0)
    segfix = cos_ref[:, TCUT:V]
    lj = lax.broadcasted_iota(jnp.int32, (RBLK, 128), 1)
    cl = jnp.sum(
        jnp.where(lj == lane, seg, 0.0), axis=1, keepdims=True
    ) + jnp.sum(
        jnp.where(lj == tlane, segfix, 0.0), axis=1, keepdims=True
    )

    t = jnp.exp2(cl * K2 - lse2)
    d = 2.0 ** (-MARGIN * SCALE * LOG2E)
    term = jnp.maximum(1.0 - t * (1.0 - d), 1e-37)
    lse2m = lse2 + jnp.log2(term)
    loss = LN2 * lse2m - SCALE * (cl - MARGIN)
    acc_ref[...] += jnp.sum(loss, axis=0, keepdims=True)

    @pl.when(i == NI - 1)
    def _finish():
        out_ref[0, :, :] = acc_ref[...] * (1.0 / B)


@functools.partial(jax.jit, static_argnames=("interpret",))
def _amloss(cosine, label, interpret=False):
    lab2d = label.reshape(B, 1).astype(jnp.int32)
    out = pl.pallas_call(
        _tc_kernel,
        grid=(GRID_R, NI),
        in_specs=[
            pl.BlockSpec((RBLK, V), lambda r, i: (r * NI + i, 0)),
            pl.BlockSpec(
                (RBLK, 1),
                lambda r, i: (r * NI + i, 0),
                memory_space=pltpu.SMEM,
            ),
        ],
        out_specs=pl.BlockSpec((1, 1, 1), lambda r, i: (r, 0, 0)),
        out_shape=jax.ShapeDtypeStruct((GRID_R, 1, 1), jnp.float32),
        scratch_shapes=[
            pltpu.VMEM((1, 1), jnp.float32),
        ],
        compiler_params=pltpu.CompilerParams(
            dimension_semantics=("parallel", "arbitrary")
        ),
        interpret=interpret,
    )(cosine, lab2d)
    return jnp.sum(out)


def kernel(cosine, label):
    return _amloss(cosine, label)
